# baseline (speedup 1.0000x reference)
import functools

import jax
import jax.numpy as jnp
from jax import lax
from jax.experimental import pallas as pl
from jax.experimental.pallas import tpu as pltpu
from jax.experimental.pallas import tpu_sc as plsc

NUM_CORES = 2
NUM_SUBCORES = 16
NUM_WORKERS = NUM_CORES * NUM_SUBCORES
LANES = 16

BATCH_N = 4096
NUM_EMB = 1000000
SEQ = 200
DIM = 64
NCLS = 3
B_PER_W = BATCH_N // NUM_WORKERS
IDS_PER_W = B_PER_W * SEQ
C1 = 128
C2 = SEQ - C1
DREG = DIM // LANES
OUT_W = LANES

TBLK = 128
NBLK = (NUM_EMB + TBLK - 1) // TBLK
NFULL = NUM_EMB // TBLK
ROWS_PAD = NBLK * TBLK
BLK_ELEMS = TBLK * DIM


def _tr_body(t3_hbm, lin_hbm, in0, in1, ob0, ob1, si0, si1, so0, so1):
    wid = lax.axis_index("s") * NUM_CORES + lax.axis_index("c")
    nblk = (NFULL - wid) // NUM_WORKERS + 1

    iota64 = lax.iota(jnp.int32, LANES) * DIM

    def fire_in(blk, buf, sem):
        rb = blk * TBLK

        @pl.when(blk < NFULL)
        def _():
            pltpu.async_copy(t3_hbm.at[:, :, pl.ds(rb, TBLK)], buf, sem)

        @pl.when(blk == NFULL)
        def _():
            pltpu.async_copy(t3_hbm.at[:, :, pl.ds(rb, TBLK // 2)],
                             buf.at[:, :, pl.ds(0, TBLK // 2)], sem)

    def drain_in(blk, buf, sem):
        @pl.when(blk < NFULL)
        def _():
            pltpu.make_async_copy(t3_hbm.at[:, :, pl.ds(0, TBLK)],
                                  buf, sem).wait()

        @pl.when(blk == NFULL)
        def _():
            pltpu.make_async_copy(t3_hbm.at[:, :, pl.ds(0, TBLK // 2)],
                                  buf.at[:, :, pl.ds(0, TBLK // 2)],
                                  sem).wait()

    def transpose(blk, buf, ob):
        def step(i, carry):
            c = i >> 3
            g = i & 7
            a = c >> 3
            s = c & 7
            v = buf[a, s, pl.ds(g * LANES, LANES)]
            plsc.store_scatter(ob, [iota64 + (g * 1024 + c)], v)
            return carry

        lax.fori_loop(0, DIM * 8, step, 0, unroll=8)

    def process(j, in_buf, si, ob, so):
        blk = wid + NUM_WORKERS * j
        drain_in(blk, in_buf, si)

        @pl.when(j >= 2)
        def _():
            pltpu.make_async_copy(t3_hbm.at[0, 0, pl.ds(0, BLK_ELEMS)],
                                  ob, so).wait()

        transpose(blk, in_buf, ob)

        nxt = j + 2
        @pl.when(nxt < nblk)
        def _():
            fire_in(wid + NUM_WORKERS * nxt, in_buf, si)

        pltpu.async_copy(ob, lin_hbm.at[pl.ds(blk * BLK_ELEMS, BLK_ELEMS)], so)

    fire_in(wid, in0, si0)
    fire_in(wid + NUM_WORKERS, in1, si1)

    def pair(p, carry):
        process(2 * p, in0, si0, ob0, so0)

        @pl.when(2 * p + 1 < nblk)
        def _():
            process(2 * p + 1, in1, si1, ob1, so1)

        return carry

    lax.fori_loop(0, (nblk + 1) // 2, pair, 0)
    pltpu.make_async_copy(t3_hbm.at[0, 0, pl.ds(0, BLK_ELEMS)], ob0, so0).wait()
    pltpu.make_async_copy(t3_hbm.at[0, 0, pl.ds(0, BLK_ELEMS)], ob1, so1).wait()


_tr_call = pl.kernel(
    _tr_body,
    out_type=jax.ShapeDtypeStruct((ROWS_PAD * DIM,), jnp.float32),
    mesh=plsc.VectorSubcoreMesh(core_axis_name="c", subcore_axis_name="s"),
    compiler_params=pltpu.CompilerParams(
        needs_layout_passes=False, use_tc_tiling_on_sc=True),
    scratch_types=[
        pltpu.VMEM((8, 8, TBLK), jnp.float32),
        pltpu.VMEM((8, 8, TBLK), jnp.float32),
        pltpu.VMEM((BLK_ELEMS,), jnp.float32),
        pltpu.VMEM((BLK_ELEMS,), jnp.float32),
        pltpu.SemaphoreType.DMA,
        pltpu.SemaphoreType.DMA,
        pltpu.SemaphoreType.DMA,
        pltpu.SemaphoreType.DMA,
    ],
)


def _sc_body(ids_hbm, table_hbm, wt_hbm, bv_hbm, out_hbm,
             idx_v, rows0_v, rows1_v, wt_v, bv_v, out_v, sem0, sem1):
    wid = lax.axis_index("s") * NUM_CORES + lax.axis_index("c")
    base = wid * B_PER_W

    pltpu.sync_copy(ids_hbm.at[pl.ds(base * SEQ, IDS_PER_W)], idx_v)
    pltpu.sync_copy(wt_hbm, wt_v)
    pltpu.sync_copy(bv_hbm, bv_v)

    w = [[wt_v[pl.ds(c * DIM + k * LANES, LANES)] for k in range(DREG)]
         for c in range(NCLS)]
    bvec = [bv_v[pl.ds(c * LANES, LANES)] for c in range(NCLS)]

    def fire(i, buf, s):
        pltpu.async_copy(table_hbm.at[idx_v.at[pl.ds(i * SEQ, C1)]],
                         buf.at[pl.ds(0, C1)], s)
        pltpu.async_copy(table_hbm.at[idx_v.at[pl.ds(i * SEQ + C1, C2)]],
                         buf.at[pl.ds(C1, C2)], s)

    def drain(buf, s):
        pltpu.make_async_copy(table_hbm.at[pl.ds(0, SEQ)], buf, s).wait()

    zero = jnp.zeros((LANES,), jnp.float32)
    lane = lax.iota(jnp.int32, LANES)

    def process(i, buf, s):
        drain(buf, s)

        def accum(t, acc):
            return tuple(acc[k] + buf[t, pl.ds(k * LANES, LANES)]
                         for k in range(DREG))

        acc = lax.fori_loop(0, SEQ, accum, (zero,) * DREG, unroll=8)

        nxt = i + 2
        @pl.when(nxt < B_PER_W)
        def _():
            fire(nxt, buf, s)

        res = zero
        for c in range(NCLS):
            t = bvec[c]
            for k in range(DREG):
                t = t + acc[k] * w[c][k]
            res = jnp.where(lane == c, jnp.full((LANES,), jnp.sum(t)), res)
        out_v[pl.ds(i * OUT_W, OUT_W)] = res

    fire(0, rows0_v, sem0)
    fire(1, rows1_v, sem1)

    def pair(j, carry):
        process(2 * j, rows0_v, sem0)
        process(2 * j + 1, rows1_v, sem1)
        return carry

    lax.fori_loop(0, B_PER_W // 2, pair, 0)
    pltpu.sync_copy(out_v, out_hbm.at[pl.ds(base * OUT_W, B_PER_W * OUT_W)])


_sc_call = pl.kernel(
    _sc_body,
    out_type=jax.ShapeDtypeStruct((BATCH_N * OUT_W,), jnp.float32),
    mesh=plsc.VectorSubcoreMesh(core_axis_name="c", subcore_axis_name="s"),
    compiler_params=pltpu.CompilerParams(
        needs_layout_passes=False, use_tc_tiling_on_sc=False),
    scratch_types=[
        pltpu.VMEM((IDS_PER_W,), jnp.int32),
        pltpu.VMEM((SEQ, DIM), jnp.float32),
        pltpu.VMEM((SEQ, DIM), jnp.float32),
        pltpu.VMEM((NCLS * DIM,), jnp.float32),
        pltpu.VMEM((NCLS * LANES,), jnp.float32),
        pltpu.VMEM((B_PER_W * OUT_W,), jnp.float32),
        pltpu.SemaphoreType.DMA,
        pltpu.SemaphoreType.DMA,
    ],
)


@jax.jit
def kernel(subword_ids, table, W, b):
    wt = (W.T / SEQ).astype(jnp.float32).reshape(-1)
    bv = jnp.broadcast_to(b[:, None] / LANES,
                          (NCLS, LANES)).astype(jnp.float32).reshape(-1)
    t3 = table.T.reshape(8, 8, NUM_EMB)
    lin = _tr_call(t3)
    out = _sc_call(subword_ids.reshape(-1),
                   lin.reshape(ROWS_PAD, DIM), wt, bv)
    return out.reshape(BATCH_N, OUT_W)[:, :NCLS]


# speedup vs baseline: 1.0587x; 1.0587x over previous
import functools

import jax
import jax.numpy as jnp
from jax import lax
from jax.experimental import pallas as pl
from jax.experimental.pallas import tpu as pltpu
from jax.experimental.pallas import tpu_sc as plsc

NUM_CORES = 2
NUM_SUBCORES = 16
NUM_WORKERS = NUM_CORES * NUM_SUBCORES
LANES = 16

BATCH_N = 4096
NUM_EMB = 1000000
SEQ = 200
DIM = 64
NCLS = 3
B_PER_W = BATCH_N // NUM_WORKERS
IDS_PER_W = B_PER_W * SEQ
C1 = 128
C2 = SEQ - C1
DREG = DIM // LANES
OUT_W = LANES

TBLK = 128
NBLK = (NUM_EMB + TBLK - 1) // TBLK
NFULL = NUM_EMB // TBLK
ROWS_PAD = NBLK * TBLK
BLK_ELEMS = TBLK * DIM


def _tr_body(t3_hbm, lin_hbm, in0, in1, ob0, ob1, si0, si1, so0, so1):
    wid = lax.axis_index("s") * NUM_CORES + lax.axis_index("c")
    nblk = (NFULL - wid) // NUM_WORKERS + 1

    iota64 = lax.iota(jnp.int32, LANES) * DIM

    def fire_in(blk, buf, sem):
        rb = blk * TBLK

        @pl.when(blk < NFULL)
        def _():
            pltpu.async_copy(t3_hbm.at[:, :, pl.ds(rb, TBLK)], buf, sem)

        @pl.when(blk == NFULL)
        def _():
            pltpu.async_copy(t3_hbm.at[:, :, pl.ds(rb, TBLK // 2)],
                             buf.at[:, :, pl.ds(0, TBLK // 2)], sem)

    def drain_in(blk, buf, sem):
        @pl.when(blk < NFULL)
        def _():
            pltpu.make_async_copy(t3_hbm.at[:, :, pl.ds(0, TBLK)],
                                  buf, sem).wait()

        @pl.when(blk == NFULL)
        def _():
            pltpu.make_async_copy(t3_hbm.at[:, :, pl.ds(0, TBLK // 2)],
                                  buf.at[:, :, pl.ds(0, TBLK // 2)],
                                  sem).wait()

    def transpose(blk, buf, ob):
        @plsc.parallel_loop(0, DIM * 8, unroll=8)
        def _(i):
            c = i >> 3
            g = i & 7
            a = c >> 3
            s = c & 7
            v = buf[a, s, pl.ds(g * LANES, LANES)]
            plsc.store_scatter(ob, [iota64 + (g * 1024 + c)], v)

    def process(j, in_buf, si, ob, so):
        blk = wid + NUM_WORKERS * j
        drain_in(blk, in_buf, si)

        @pl.when(j >= 2)
        def _():
            pltpu.make_async_copy(t3_hbm.at[0, 0, pl.ds(0, BLK_ELEMS)],
                                  ob, so).wait()

        transpose(blk, in_buf, ob)

        nxt = j + 2
        @pl.when(nxt < nblk)
        def _():
            fire_in(wid + NUM_WORKERS * nxt, in_buf, si)

        pltpu.async_copy(ob, lin_hbm.at[pl.ds(blk * BLK_ELEMS, BLK_ELEMS)], so)

    fire_in(wid, in0, si0)
    fire_in(wid + NUM_WORKERS, in1, si1)

    def pair(p, carry):
        process(2 * p, in0, si0, ob0, so0)

        @pl.when(2 * p + 1 < nblk)
        def _():
            process(2 * p + 1, in1, si1, ob1, so1)

        return carry

    lax.fori_loop(0, (nblk + 1) // 2, pair, 0)
    pltpu.make_async_copy(t3_hbm.at[0, 0, pl.ds(0, BLK_ELEMS)], ob0, so0).wait()
    pltpu.make_async_copy(t3_hbm.at[0, 0, pl.ds(0, BLK_ELEMS)], ob1, so1).wait()


_tr_call = pl.kernel(
    _tr_body,
    out_type=jax.ShapeDtypeStruct((ROWS_PAD * DIM,), jnp.float32),
    mesh=plsc.VectorSubcoreMesh(core_axis_name="c", subcore_axis_name="s"),
    compiler_params=pltpu.CompilerParams(
        needs_layout_passes=False, use_tc_tiling_on_sc=True),
    scratch_types=[
        pltpu.VMEM((8, 8, TBLK), jnp.float32),
        pltpu.VMEM((8, 8, TBLK), jnp.float32),
        pltpu.VMEM((BLK_ELEMS,), jnp.float32),
        pltpu.VMEM((BLK_ELEMS,), jnp.float32),
        pltpu.SemaphoreType.DMA,
        pltpu.SemaphoreType.DMA,
        pltpu.SemaphoreType.DMA,
        pltpu.SemaphoreType.DMA,
    ],
)


def _sc_body(ids_hbm, table_hbm, wt_hbm, bv_hbm, out_hbm,
             idx_v, rows0_v, rows1_v, wt_v, bv_v, out_v, sem0, sem1):
    wid = lax.axis_index("s") * NUM_CORES + lax.axis_index("c")
    base = wid * B_PER_W

    pltpu.sync_copy(ids_hbm.at[pl.ds(base * SEQ, IDS_PER_W)], idx_v)
    pltpu.sync_copy(wt_hbm, wt_v)
    pltpu.sync_copy(bv_hbm, bv_v)

    w = [[wt_v[pl.ds(c * DIM + k * LANES, LANES)] for k in range(DREG)]
         for c in range(NCLS)]
    bvec = [bv_v[pl.ds(c * LANES, LANES)] for c in range(NCLS)]

    def fire(i, buf, s):
        pltpu.async_copy(table_hbm.at[idx_v.at[pl.ds(i * SEQ, C1)]],
                         buf.at[pl.ds(0, C1)], s)
        pltpu.async_copy(table_hbm.at[idx_v.at[pl.ds(i * SEQ + C1, C2)]],
                         buf.at[pl.ds(C1, C2)], s)

    def drain(buf, s):
        pltpu.make_async_copy(table_hbm.at[pl.ds(0, SEQ)], buf, s).wait()

    zero = jnp.zeros((LANES,), jnp.float32)
    lane = lax.iota(jnp.int32, LANES)

    def process(i, buf, s):
        drain(buf, s)

        def accum(t, acc):
            return tuple(acc[k] + buf[t, pl.ds(k * LANES, LANES)]
                         for k in range(DREG))

        acc = lax.fori_loop(0, SEQ, accum, (zero,) * DREG, unroll=8)

        nxt = i + 2
        @pl.when(nxt < B_PER_W)
        def _():
            fire(nxt, buf, s)

        res = zero
        for c in range(NCLS):
            t = bvec[c]
            for k in range(DREG):
                t = t + acc[k] * w[c][k]
            res = jnp.where(lane == c, jnp.full((LANES,), jnp.sum(t)), res)
        out_v[pl.ds(i * OUT_W, OUT_W)] = res

    fire(0, rows0_v, sem0)
    fire(1, rows1_v, sem1)

    def pair(j, carry):
        process(2 * j, rows0_v, sem0)
        process(2 * j + 1, rows1_v, sem1)
        return carry

    lax.fori_loop(0, B_PER_W // 2, pair, 0)
    pltpu.sync_copy(out_v, out_hbm.at[pl.ds(base * OUT_W, B_PER_W * OUT_W)])


_sc_call = pl.kernel(
    _sc_body,
    out_type=jax.ShapeDtypeStruct((BATCH_N * OUT_W,), jnp.float32),
    mesh=plsc.VectorSubcoreMesh(core_axis_name="c", subcore_axis_name="s"),
    compiler_params=pltpu.CompilerParams(
        needs_layout_passes=False, use_tc_tiling_on_sc=False),
    scratch_types=[
        pltpu.VMEM((IDS_PER_W,), jnp.int32),
        pltpu.VMEM((SEQ, DIM), jnp.float32),
        pltpu.VMEM((SEQ, DIM), jnp.float32),
        pltpu.VMEM((NCLS * DIM,), jnp.float32),
        pltpu.VMEM((NCLS * LANES,), jnp.float32),
        pltpu.VMEM((B_PER_W * OUT_W,), jnp.float32),
        pltpu.SemaphoreType.DMA,
        pltpu.SemaphoreType.DMA,
    ],
)


@jax.jit
def kernel(subword_ids, table, W, b):
    wt = (W.T / SEQ).astype(jnp.float32).reshape(-1)
    bv = jnp.broadcast_to(b[:, None] / LANES,
                          (NCLS, LANES)).astype(jnp.float32).reshape(-1)
    t3 = table.T.reshape(8, 8, NUM_EMB)
    lin = _tr_call(t3)
    out = _sc_call(subword_ids.reshape(-1),
                   lin.reshape(ROWS_PAD, DIM), wt, bv)
    return out.reshape(BATCH_N, OUT_W)[:, :NCLS]


# speedup vs baseline: 1.8479x; 1.7455x over previous
import functools

import jax
import jax.numpy as jnp
from jax import lax
from jax.experimental import pallas as pl
from jax.experimental.pallas import tpu as pltpu
from jax.experimental.pallas import tpu_sc as plsc

NUM_CORES = 2
NUM_SUBCORES = 16
NUM_WORKERS = NUM_CORES * NUM_SUBCORES
LANES = 16

BATCH_N = 4096
NUM_EMB = 1000000
SEQ = 200
DIM = 64
NCLS = 3
B_PER_W = BATCH_N // NUM_WORKERS
IDS_PER_W = B_PER_W * SEQ
C1 = 128
C2 = SEQ - C1
DREG = DIM // LANES
OUT_W = LANES


def _sc_body(ids_hbm, table_hbm, wt_hbm, bv_hbm, out_hbm,
             idx_v, rows0_v, rows1_v, wt_v, bv_v, out_v, sem0, sem1):
    wid = lax.axis_index("s") * NUM_CORES + lax.axis_index("c")
    base = wid * B_PER_W

    pltpu.sync_copy(ids_hbm.at[pl.ds(base * SEQ, IDS_PER_W)], idx_v)
    pltpu.sync_copy(wt_hbm, wt_v)
    pltpu.sync_copy(bv_hbm, bv_v)

    w = [[wt_v[pl.ds(c * DIM + k * LANES, LANES)] for k in range(DREG)]
         for c in range(NCLS)]
    bvec = [bv_v[pl.ds(c * LANES, LANES)] for c in range(NCLS)]

    def fire(i, buf, s):
        pltpu.async_copy(table_hbm.at[idx_v.at[pl.ds(i * SEQ, C1)]],
                         buf.at[pl.ds(0, C1)], s)
        pltpu.async_copy(table_hbm.at[idx_v.at[pl.ds(i * SEQ + C1, C2)]],
                         buf.at[pl.ds(C1, C2)], s)

    def drain(buf, s):
        pltpu.make_async_copy(table_hbm.at[pl.ds(0, SEQ)], buf, s).wait()

    zero = jnp.zeros((LANES,), jnp.float32)
    lane = lax.iota(jnp.int32, LANES)

    def process(i, buf, s):
        drain(buf, s)

        def accum(t, acc):
            return tuple(acc[k] + buf[t, pl.ds(k * LANES, LANES)]
                         for k in range(DREG))

        acc = lax.fori_loop(0, SEQ, accum, (zero,) * DREG, unroll=8)

        nxt = i + 2
        @pl.when(nxt < B_PER_W)
        def _():
            fire(nxt, buf, s)

        res = zero
        for c in range(NCLS):
            t = bvec[c]
            for k in range(DREG):
                t = t + acc[k] * w[c][k]
            res = jnp.where(lane == c, jnp.full((LANES,), jnp.sum(t)), res)
        out_v[pl.ds(i * OUT_W, OUT_W)] = res

    fire(0, rows0_v, sem0)
    fire(1, rows1_v, sem1)

    def pair(j, carry):
        process(2 * j, rows0_v, sem0)
        process(2 * j + 1, rows1_v, sem1)
        return carry

    lax.fori_loop(0, B_PER_W // 2, pair, 0)
    pltpu.sync_copy(out_v, out_hbm.at[pl.ds(base * OUT_W, B_PER_W * OUT_W)])


_sc_call = pl.kernel(
    _sc_body,
    out_type=jax.ShapeDtypeStruct((BATCH_N * OUT_W,), jnp.float32),
    mesh=plsc.VectorSubcoreMesh(core_axis_name="c", subcore_axis_name="s"),
    compiler_params=pltpu.CompilerParams(
        needs_layout_passes=False, use_tc_tiling_on_sc=False),
    scratch_types=[
        pltpu.VMEM((IDS_PER_W,), jnp.int32),
        pltpu.VMEM((SEQ, DIM), jnp.float32),
        pltpu.VMEM((SEQ, DIM), jnp.float32),
        pltpu.VMEM((NCLS * DIM,), jnp.float32),
        pltpu.VMEM((NCLS * LANES,), jnp.float32),
        pltpu.VMEM((B_PER_W * OUT_W,), jnp.float32),
        pltpu.SemaphoreType.DMA,
        pltpu.SemaphoreType.DMA,
    ],
)


@jax.jit
def kernel(subword_ids, table, W, b):
    wt = (W.T / SEQ).astype(jnp.float32).reshape(-1)
    bv = jnp.broadcast_to(b[:, None] / LANES,
                          (NCLS, LANES)).astype(jnp.float32).reshape(-1)
    out = _sc_call(subword_ids.reshape(-1), table, wt, bv)
    return out.reshape(BATCH_N, OUT_W)[:, :NCLS]
